# baseline (speedup 1.0000x reference)
import functools

import jax
import jax.numpy as jnp
from jax import lax
from jax.experimental import pallas as pl
from jax.experimental.pallas import tpu as pltpu
from jax.experimental.pallas import tpu_sc as plsc

B = 16384
D = 16
NEG = 4
NC = 2
NS = 16
NW = NC * NS
BPW = B // NW
NPW = NEG * BPW

_mesh = plsc.VectorSubcoreMesh(
    core_axis_name="c", subcore_axis_name="s", num_cores=NC, num_subcores=NS
)


@functools.partial(
    pl.kernel,
    out_type=(
        jax.ShapeDtypeStruct((B, D), jnp.float32),
        jax.ShapeDtypeStruct((B, D), jnp.float32),
        jax.ShapeDtypeStruct((NEG * B, D), jnp.float32),
    ),
    mesh=_mesh,
    compiler_params=pltpu.CompilerParams(use_tc_tiling_on_sc=False),
    scratch_types=[
        pltpu.VMEM((BPW,), jnp.int32),
        pltpu.VMEM((BPW,), jnp.int32),
        pltpu.VMEM((NPW,), jnp.int32),
        pltpu.VMEM((BPW, D), jnp.float32),
        pltpu.VMEM((BPW, D), jnp.float32),
        pltpu.VMEM((NPW, D), jnp.float32),
        pltpu.SemaphoreType.DMA,
        pltpu.SemaphoreType.DMA,
        pltpu.SemaphoreType.DMA,
    ],
)
def _sc_gather(uid_hbm, pid_hbm, nid_hbm, ut_hbm, it_hbm,
               ue_out, pe_out, ne_out,
               idx_u, idx_p, idx_n, rows_u, rows_p, rows_n, s1, s2, s3):
    wid = lax.axis_index("c") * NS + lax.axis_index("s")
    ub = wid * BPW
    nb = wid * NPW
    pltpu.sync_copy(uid_hbm.at[pl.ds(ub, BPW)], idx_u)
    pltpu.sync_copy(pid_hbm.at[pl.ds(ub, BPW)], idx_p)
    pltpu.sync_copy(nid_hbm.at[pl.ds(nb, NPW)], idx_n)
    cu = pltpu.async_copy(ut_hbm.at[idx_u], rows_u, s1)
    cp = pltpu.async_copy(it_hbm.at[idx_p], rows_p, s2)
    cn = pltpu.async_copy(it_hbm.at[idx_n], rows_n, s3)
    cu.wait()
    cp.wait()
    cn.wait()
    pltpu.sync_copy(rows_u, ue_out.at[pl.ds(ub, BPW)])
    pltpu.sync_copy(rows_p, pe_out.at[pl.ds(ub, BPW)])
    pltpu.sync_copy(rows_n, ne_out.at[pl.ds(nb, NPW)])


BLK = 2048


def _tc_loss_body(ue_ref, pe_ref, ne_ref, out_ref):
    u = ue_ref[...]
    su = jnp.sum(u * u, axis=1, keepdims=True)
    ru = jnp.sqrt(su)
    acc = None
    for k in range(NEG + 1):
        c = pe_ref[...] if k == 0 else ne_ref[:, k - 1, :]
        dot = jnp.sum(u * c, axis=1, keepdims=True)
        sc2 = jnp.sum(c * c, axis=1, keepdims=True)
        denom = jnp.sqrt(sc2) * ru + 1e-6
        d = (dot / denom + 1.0) * 0.5
        term = jnp.log(d + 1e-6) if k == 0 else jnp.log(1.0 - d + 1e-6)
        acc = term if acc is None else acc + term

    @pl.when(pl.program_id(0) == 0)
    def _():
        out_ref[0, 0] = 0.0

    out_ref[0, 0] += -jnp.sum(acc) / B


_tc_loss = pl.pallas_call(
    _tc_loss_body,
    grid=(B // BLK,),
    in_specs=[
        pl.BlockSpec((BLK, D), lambda i: (i, 0)),
        pl.BlockSpec((BLK, D), lambda i: (i, 0)),
        pl.BlockSpec((BLK, NEG, D), lambda i: (i, 0, 0)),
    ],
    out_shape=jax.ShapeDtypeStruct((1, 1), jnp.float32),
    out_specs=pl.BlockSpec(memory_space=pltpu.SMEM),
)


def kernel(userid, itemid, user_feature, item_feature, neg_sample,
           user_table, item_table):
    uid = userid.reshape(B).astype(jnp.int32)
    pid = itemid.reshape(B).astype(jnp.int32)
    nid = neg_sample.reshape(NEG * B).astype(jnp.int32)
    ue, pe, ne = _sc_gather(uid, pid, nid, user_table, item_table)
    return _tc_loss(ue, pe, ne.reshape(B, NEG, D))[0, 0]


# speedup vs baseline: 1.0435x; 1.0435x over previous
import functools

import jax
import jax.numpy as jnp
from jax import lax
from jax.experimental import pallas as pl
from jax.experimental.pallas import tpu as pltpu
from jax.experimental.pallas import tpu_sc as plsc

B = 16384
D = 16
NEG = 4
V = 1000000
NC = 2
NS = 16
NW = NC * NS
BPW = B // NW
NPW = (NEG * B) // NW

_mesh = plsc.VectorSubcoreMesh(
    core_axis_name="c", subcore_axis_name="s", num_cores=NC, num_subcores=NS
)


@functools.partial(
    pl.kernel,
    out_type=(
        jax.ShapeDtypeStruct((D, B), jnp.float32),
        jax.ShapeDtypeStruct((D, B), jnp.float32),
        jax.ShapeDtypeStruct((NEG, D, B), jnp.float32),
    ),
    mesh=_mesh,
    compiler_params=pltpu.CompilerParams(use_tc_tiling_on_sc=False),
    scratch_types=[
        pltpu.VMEM((BPW,), jnp.int32),
        pltpu.VMEM((BPW,), jnp.int32),
        pltpu.VMEM((NPW,), jnp.int32),
        pltpu.VMEM((NPW,), jnp.int32),
        pltpu.VMEM((D, NPW), jnp.int32),
        pltpu.VMEM((D, BPW), jnp.float32),
        pltpu.VMEM((D, BPW), jnp.float32),
        pltpu.VMEM((D, NPW), jnp.float32),
        pltpu.SemaphoreType.DMA,
        pltpu.SemaphoreType.DMA,
        pltpu.SemaphoreType.DMA,
    ],
)
def _sc_gather(uid_hbm, pid_hbm, nid_hbm, ut_hbm, it_hbm,
               ue_out, pe_out, ne_out,
               idx_u, idx_p, idx_q, idx_n, idxmat, rows_u, rows_p, rows_n,
               s1, s2, s3):
    wid = lax.axis_index("c") * NS + lax.axis_index("s")
    ub = wid * BPW
    lane16 = lax.iota(jnp.int32, 16)

    pltpu.sync_copy(uid_hbm.at[pl.ds(ub, BPW)], idx_u)
    pltpu.sync_copy(pid_hbm.at[pl.ds(ub, BPW)], idx_p)

    jj = wid // 8
    b0 = (wid % 8) * NPW

    def qbody(g, _):
        q = (b0 * 4 + jj * 128) + (g // 8) * 512 + (g % 8) * 16
        idx_q[pl.ds(g * 16, 16)] = lane16 + q
        return _

    lax.fori_loop(0, NPW // 16, qbody, None)
    pltpu.async_copy(nid_hbm.at[idx_q], idx_n, s3).wait()

    def gather_planes(idx_ref, n, tbl, rows, sem):
        def body(g, _):
            v = idx_ref[pl.ds(g * 16, 16)] * 16
            for d in range(D):
                idxmat[d, pl.ds(g * 16, 16)] = v + d
            return _

        lax.fori_loop(0, n // 16, body, None)
        copies = [
            pltpu.async_copy(
                tbl.at[idxmat.at[d, pl.ds(0, n)]], rows.at[d], sem
            )
            for d in range(D)
        ]
        for cp in copies:
            cp.wait()

    gather_planes(idx_u, BPW, ut_hbm, rows_u, s1)
    pltpu.sync_copy(rows_u, ue_out.at[:, pl.ds(ub, BPW)])
    gather_planes(idx_p, BPW, it_hbm, rows_p, s2)
    pltpu.sync_copy(rows_p, pe_out.at[:, pl.ds(ub, BPW)])
    gather_planes(idx_n, NPW, it_hbm, rows_n, s3)
    pltpu.sync_copy(rows_n, ne_out.at[jj, :, pl.ds(b0, NPW)])


def _tc_loss_body(ue_ref, pe_ref, ne_ref, out_ref):
    u = ue_ref[...]
    su = jnp.sum(u * u, axis=0, keepdims=True)
    ru = jnp.sqrt(su)
    acc = None
    for k in range(NEG + 1):
        v = pe_ref[...] if k == 0 else ne_ref[k - 1]
        dot = jnp.sum(u * v, axis=0, keepdims=True)
        sv = jnp.sum(v * v, axis=0, keepdims=True)
        denom = jnp.sqrt(sv) * ru + 1e-6
        d = (dot / denom + 1.0) * 0.5
        term = jnp.log(d + 1e-6) if k == 0 else jnp.log(1.0 - d + 1e-6)
        acc = term if acc is None else acc + term
    out_ref[0, 0] = -jnp.sum(acc) / B


_tc_loss = pl.pallas_call(
    _tc_loss_body,
    out_shape=jax.ShapeDtypeStruct((1, 1), jnp.float32),
    out_specs=pl.BlockSpec(memory_space=pltpu.SMEM),
)


def kernel(userid, itemid, user_feature, item_feature, neg_sample,
           user_table, item_table):
    uid = userid.reshape(B).astype(jnp.int32)
    pid = itemid.reshape(B).astype(jnp.int32)
    nid = neg_sample.reshape(128, 128, NEG).transpose(0, 2, 1).reshape(NEG * B)
    ue, pe, ne = _sc_gather(uid, pid, nid,
                            user_table.reshape(V * D),
                            item_table.reshape(V * D))
    return _tc_loss(ue, pe, ne)[0, 0]
